# baseline (speedup 1.0000x reference)
import jax
import jax.numpy as jnp
from jax import lax
from jax.experimental import pallas as pl
from jax.experimental.pallas import tpu as pltpu
from jax.experimental.pallas import tpu_sc as plsc

EMBEDDING_DIM = 128
PACKED_DIM = EMBEDDING_DIM // 2
WINDOW = 128
NBUF = 4
NUM_CORES = 2
NUM_SUBCORES = 16
NUM_WORKERS = NUM_CORES * NUM_SUBCORES
NUM_CHUNKS = 5
CONV_BLOCK = 2048


def _sc_gather_packed(table32, idx2d):
    num_windows = idx2d.shape[0]
    steps_per_worker = num_windows // NUM_WORKERS
    idx3d = idx2d.reshape(NUM_WORKERS, steps_per_worker, WINDOW)

    mesh = plsc.VectorSubcoreMesh(
        core_axis_name="core", subcore_axis_name="subcore"
    )

    @pl.kernel(
        out_type=jax.ShapeDtypeStruct(
            (num_windows, WINDOW, PACKED_DIM), jnp.int32
        ),
        mesh=mesh,
        compiler_params=pltpu.CompilerParams(use_tc_tiling_on_sc=False),
        scratch_types=[
            pltpu.VMEM((steps_per_worker, WINDOW), jnp.int32),
            pltpu.VMEM((NBUF, WINDOW, PACKED_DIM), jnp.int32),
            pltpu.SemaphoreType.DMA((NBUF,)),
            pltpu.SemaphoreType.DMA((NBUF // 2,)),
        ],
    )
    def gather_kernel(table_hbm, idx_hbm, out_hbm, idx_v, bufs, gsem, osem):
        wid = lax.axis_index("subcore") * NUM_CORES + lax.axis_index("core")
        row0 = wid * steps_per_worker

        pltpu.sync_copy(idx_hbm.at[wid], idx_v)

        for b in range(NBUF):
            pltpu.async_copy(table_hbm.at[idx_v.at[b]], bufs.at[b], gsem.at[b])

        def pair_out(p, j):
            for q in range(2):
                pltpu.make_async_copy(
                    table_hbm.at[idx_v.at[j + q]],
                    bufs.at[2 * p + q],
                    gsem.at[2 * p + q],
                ).wait()
            pltpu.async_copy(
                bufs.at[pl.ds(2 * p, 2)],
                out_hbm.at[pl.ds(row0 + j, 2)],
                osem.at[p],
            )

        def pair_out_wait(p, j):
            pltpu.make_async_copy(
                bufs.at[pl.ds(2 * p, 2)],
                out_hbm.at[pl.ds(row0 + j, 2)],
                osem.at[p],
            ).wait()

        @pl.loop(0, steps_per_worker - NBUF, step=NBUF)
        def _(jo):
            for p in range(NBUF // 2):
                j = jo + 2 * p
                pair_out(p, j)
                pair_out_wait(p, j)
                for q in range(2):
                    pltpu.async_copy(
                        table_hbm.at[idx_v.at[j + NBUF + q]],
                        bufs.at[2 * p + q],
                        gsem.at[2 * p + q],
                    )

        jt = steps_per_worker - NBUF
        for p in range(NBUF // 2):
            pair_out(p, jt + 2 * p)
        for p in range(NBUF // 2):
            pair_out_wait(p, jt + 2 * p)

    return gather_kernel(table32, idx3d)


def _tc_unpack_chunk(big_f32, chunk_i32, chunk_idx, total_rows):
    rows = chunk_i32.shape[0]
    grid = rows // CONV_BLOCK
    row_off = chunk_idx * grid

    def body(_, in_ref, out_ref):
        x = in_ref[...]
        out_ref[:, :PACKED_DIM] = lax.bitcast_convert_type(
            lax.shift_left(x, 16), jnp.float32
        )
        out_ref[:, PACKED_DIM:] = lax.bitcast_convert_type(
            lax.bitwise_and(x, jnp.int32(-65536)), jnp.float32
        )

    operands = [
        big_f32
        if big_f32 is not None
        else jnp.zeros((8, EMBEDDING_DIM), jnp.float32),
        chunk_i32,
    ]
    return pl.pallas_call(
        body,
        grid=(grid,),
        in_specs=[
            pl.BlockSpec(memory_space=pl.ANY),
            pl.BlockSpec((CONV_BLOCK, PACKED_DIM), lambda i: (i, 0)),
        ],
        out_specs=pl.BlockSpec(
            (CONV_BLOCK, EMBEDDING_DIM), lambda i: (row_off + i, 0)
        ),
        out_shape=jax.ShapeDtypeStruct((total_rows, EMBEDDING_DIM), jnp.float32),
        input_output_aliases={0: 0} if big_f32 is not None else {},
    )(*operands)


def kernel(sentences_indices, embedding_table):
    batch, hist = sentences_indices.shape
    num_indices = batch * hist
    num_windows = num_indices // WINDOW
    idx2d = sentences_indices.reshape(num_windows, WINDOW).astype(jnp.int32)

    t16 = embedding_table.astype(jnp.bfloat16)
    table32 = lax.bitcast_convert_type(
        jnp.stack([t16[:, :PACKED_DIM], t16[:, PACKED_DIM:]], axis=-1),
        jnp.int32,
    )

    win_per_chunk = num_windows // NUM_CHUNKS
    out = None
    for k in range(NUM_CHUNKS):
        g32 = _sc_gather_packed(
            table32, idx2d[k * win_per_chunk : (k + 1) * win_per_chunk]
        )
        out = _tc_unpack_chunk(
            out,
            g32.reshape(win_per_chunk * WINDOW, PACKED_DIM),
            k,
            num_indices,
        )

    return out.reshape(batch, hist, EMBEDDING_DIM)


# speedup vs baseline: 3.2405x; 3.2405x over previous
import jax
import jax.numpy as jnp
from jax import lax
from jax.experimental import pallas as pl
from jax.experimental.pallas import tpu as pltpu
from jax.experimental.pallas import tpu_sc as plsc

EMBEDDING_DIM = 128
WINDOW = 128
NBUF = 4
NUM_CORES = 2
NUM_SUBCORES = 16
NUM_WORKERS = NUM_CORES * NUM_SUBCORES


def kernel(sentences_indices, embedding_table):
    batch, hist = sentences_indices.shape
    num_indices = batch * hist
    steps_per_worker = num_indices // (NUM_WORKERS * WINDOW)
    idx2d = sentences_indices.reshape(num_indices // WINDOW, WINDOW).astype(
        jnp.int32
    )

    mesh = plsc.VectorSubcoreMesh(
        core_axis_name="core", subcore_axis_name="subcore"
    )

    @pl.kernel(
        out_type=jax.ShapeDtypeStruct(
            (num_indices // WINDOW, WINDOW, EMBEDDING_DIM), jnp.float32
        ),
        mesh=mesh,
        scratch_types=[
            pltpu.VMEM((steps_per_worker, WINDOW), jnp.int32),
            pltpu.VMEM((NBUF, WINDOW, EMBEDDING_DIM), jnp.float32),
            pltpu.SemaphoreType.DMA((NBUF,)),
            pltpu.SemaphoreType.DMA((NBUF // 2,)),
        ],
    )
    def gather_kernel(table_hbm, idx_hbm, out_hbm, idx_v, bufs, gsem, osem):
        wid = lax.axis_index("subcore") * NUM_CORES + lax.axis_index("core")
        row0 = wid * steps_per_worker

        pltpu.sync_copy(idx_hbm.at[pl.ds(row0, steps_per_worker)], idx_v)

        for b in range(NBUF):
            pltpu.async_copy(table_hbm.at[idx_v.at[b]], bufs.at[b], gsem.at[b])

        def pair_out(p, j):
            for q in range(2):
                pltpu.make_async_copy(
                    table_hbm.at[idx_v.at[j + q]],
                    bufs.at[2 * p + q],
                    gsem.at[2 * p + q],
                ).wait()
            pltpu.async_copy(
                bufs.at[pl.ds(2 * p, 2)],
                out_hbm.at[pl.ds(row0 + j, 2)],
                osem.at[p],
            )

        def pair_out_wait(p, j):
            pltpu.make_async_copy(
                bufs.at[pl.ds(2 * p, 2)],
                out_hbm.at[pl.ds(row0 + j, 2)],
                osem.at[p],
            ).wait()

        @pl.loop(0, steps_per_worker - NBUF, step=NBUF)
        def _(jo):
            for p in range(NBUF // 2):
                j = jo + 2 * p
                pair_out(p, j)
                pair_out_wait(p, j)
                for q in range(2):
                    pltpu.async_copy(
                        table_hbm.at[idx_v.at[j + NBUF + q]],
                        bufs.at[2 * p + q],
                        gsem.at[2 * p + q],
                    )

        jt = steps_per_worker - NBUF
        for p in range(NBUF // 2):
            pair_out(p, jt + 2 * p)
        for p in range(NBUF // 2):
            pair_out_wait(p, jt + 2 * p)

    out = gather_kernel(embedding_table, idx2d)
    return out.reshape(batch, hist, EMBEDDING_DIM)
